# baseline (speedup 1.0000x reference)
import functools

import jax
import jax.numpy as jnp
from jax import lax
from jax.experimental import pallas as pl
from jax.experimental.pallas import tpu as pltpu
from jax.experimental.pallas import tpu_sc as plsc

N_NODES = 10000
E_EDGES = 320000
H = 128
SEQ_NUM = 12
SEQ_LEN = 12
B = 16
F = 144
NSC = 2
NSUB = 16
EPT = E_EDGES // NSUB
EB = 4000
NBLK = EPT // EB
CK = 80
NCHUNK = EB // CK
NBUF = 2
GA = 2
CKA = GA * 16
CKB = CK - CKA
NPAD = 10240
RPT = NPAD // NSUB
NVR = F // 16


_GD = lax.GatherDimensionNumbers(
    offset_dims=(), collapsed_slice_dims=(0,), start_index_map=(0,))


def _lane_bcast(vec, e):
    idx = jnp.full((16, 1), e, jnp.int32)
    return lax.gather(vec, idx, _GD, (1,),
                      mode=lax.GatherScatterMode.PROMISE_IN_BOUNDS)


def _sc_segment_sum(x_split, src, dst, w, s_index, S_emb):
    mesh = plsc.VectorSubcoreMesh(core_axis_name="c", subcore_axis_name="s")

    @functools.partial(
        pl.kernel,
        mesh=mesh,
        compiler_params=pltpu.CompilerParams(use_tc_tiling_on_sc=False),
        out_type=[
            jax.ShapeDtypeStruct((NSC * NPAD, F), jnp.float32),
            jax.ShapeDtypeStruct((B, H), jnp.float32),
        ],
        scratch_types=[
            pltpu.VMEM_SHARED((NPAD, F), jnp.float32),
            pltpu.VMEM((EB,), jnp.int32),
            pltpu.VMEM((EB,), jnp.int32),
            pltpu.VMEM((EB,), jnp.float32),
            [pltpu.VMEM((CK,), jnp.int32) for _ in range(NBUF)],
            [pltpu.VMEM((CKA,), jnp.int32) for _ in range(NBUF)],
            [pltpu.VMEM((CKB,), jnp.int32) for _ in range(NBUF)],
            [pltpu.VMEM((CK, F), jnp.float32) for _ in range(NBUF)],
            pltpu.VMEM((B,), jnp.int32),
            pltpu.VMEM((B, H), jnp.float32),
            pltpu.SemaphoreType.DMA,
            [pltpu.SemaphoreType.DMA for _ in range(NBUF)],
            [pltpu.SemaphoreType.DMA for _ in range(NBUF)],
            [pltpu.SemaphoreType.DMA for _ in range(NBUF)],
        ],
    )
    def k(x_hbm, src_hbm, dst_hbm, w_hbm, sidx_hbm, semb_hbm,
          out_hbm, sout_hbm,
          acc, srcv, dstv, wv, gidxs, dchsA, dchsB, rowss,
          sibuf, sebuf, sem, gsems, ssemsA, ssemsB):
        cid = lax.axis_index("c")
        sid = lax.axis_index("s")
        rows0 = rowss[0]

        zv = jnp.zeros((16,), jnp.float32)
        for r in range(CK):
            for j in range(NVR):
                rows0[r, pl.ds(j * 16, 16)] = zv
        for t in range(RPT // CK):
            pltpu.sync_copy(rows0, acc.at[pl.ds(sid * RPT + t * CK, CK)])
        plsc.subcore_barrier()

        gath_off = cid * N_NODES

        def prep(c, gidx, dchA, dchB):
            cb = c * CK
            for j in range(CK // 16):
                sl = pl.ds(cb + j * 16, 16)
                gidx[pl.ds(j * 16, 16)] = srcv[sl] + gath_off
                if j < GA:
                    dchA[pl.ds(j * 16, 16)] = dstv[sl]
                else:
                    dchB[pl.ds((j - GA) * 16, 16)] = dstv[sl]

        def scale(c, rows, g_lo, g_hi):
            cb = c * CK
            for g in range(g_lo, g_hi):
                w16 = wv[pl.ds(cb + g * 16, 16)]
                for e in range(16):
                    wvec = _lane_bcast(w16, e)
                    for j in range(NVR):
                        sl = pl.ds(j * 16, 16)
                        rows[g * 16 + e, sl] = rows[g * 16 + e, sl] * wvec

        def block_body(b, carry):
            ebase = sid * EPT + b * EB
            pltpu.sync_copy(src_hbm.at[pl.ds(ebase, EB)], srcv)
            pltpu.sync_copy(dst_hbm.at[pl.ds(ebase, EB)], dstv)
            pltpu.sync_copy(w_hbm.at[pl.ds(ebase, EB)], wv)

            def ring_body(i, carry2):
                started = (b > 0) | (i > 0)
                gs = []
                for u in range(NBUF):
                    c = NBUF * i + u

                    @pl.when(started)
                    def _(u=u):
                        pltpu.make_async_copy(
                            rowss[u].at[pl.ds(0, CKA)],
                            acc.at[dchsA[u]], ssemsA[u]).wait()
                        pltpu.make_async_copy(
                            rowss[u].at[pl.ds(CKA, CKB)],
                            acc.at[dchsB[u]], ssemsB[u]).wait()

                    prep(c, gidxs[u], dchsA[u], dchsB[u])
                    gs.append(pltpu.async_copy(
                        x_hbm.at[gidxs[u]], rowss[u], gsems[u]))
                for u in range(NBUF):
                    c = NBUF * i + u
                    gs[u].wait()
                    scale(c, rowss[u], 0, GA)
                    pltpu.async_copy(rowss[u].at[pl.ds(0, CKA)],
                                     acc.at[dchsA[u]], ssemsA[u], add=True)
                    scale(c, rowss[u], GA, CK // 16)
                    pltpu.async_copy(rowss[u].at[pl.ds(CKA, CKB)],
                                     acc.at[dchsB[u]], ssemsB[u], add=True)
                return carry2

            lax.fori_loop(0, NCHUNK // NBUF, ring_body, 0)
            return carry

        lax.fori_loop(0, NBLK, block_body, 0)
        for u in range(NBUF):
            pltpu.make_async_copy(rowss[u].at[pl.ds(0, CKA)],
                                  acc.at[dchsA[u]], ssemsA[u]).wait()
            pltpu.make_async_copy(rowss[u].at[pl.ds(CKA, CKB)],
                                  acc.at[dchsB[u]], ssemsB[u]).wait()
        plsc.subcore_barrier()

        out_off = cid * NPAD
        for t in range(RPT // CK):
            rb = sid * RPT + t * CK
            pltpu.sync_copy(acc.at[pl.ds(rb, CK)], rows0)
            pltpu.sync_copy(rows0, out_hbm.at[pl.ds(out_off + rb, CK)])

        @pl.when(jnp.logical_and(cid == 0, sid == 0))
        def _():
            pltpu.sync_copy(sidx_hbm, sibuf)
            pltpu.async_copy(semb_hbm.at[sibuf], sebuf, sem).wait()
            pltpu.sync_copy(sebuf, sout_hbm)

    return k(x_split, src, dst, w, s_index, S_emb)


BLK = 512
GRID = NPAD // BLK


def _tc_body(agghp_ref, aggcp_ref, batch_ref, idx_ref, oh43_ref, semb_ref,
             z_ref, Wg_ref, bg_ref, Wst_ref, bst_ref, Wfc_ref, bfc_ref,
             Wout_ref, bout_ref, out_ref, acc_c, acc_h, acc_o, acc_n):
    i = pl.program_id(0)

    @pl.when(i == 0)
    def _init():
        acc_c[...] = jnp.zeros_like(acc_c)
        acc_h[...] = jnp.zeros_like(acc_h)
        acc_o[...] = jnp.zeros_like(acc_o)
        acc_n[...] = jnp.zeros_like(acc_n)

    f32 = jnp.float32
    hc = jnp.tanh(
        jnp.dot(aggcp_ref[:, :H], Wg_ref[...], preferred_element_type=f32)
        + bg_ref[...])
    aggh = agghp_ref[...]
    Wst = Wst_ref[...]
    bst = bst_ref[...]
    hsum = jnp.zeros((BLK, H), f32)
    h_last = None
    for t in range(SEQ_NUM):
        ht = jnp.tanh(
            jnp.dot(aggh[:, t * SEQ_LEN:(t + 1) * SEQ_LEN], Wst,
                    preferred_element_type=f32) + bst)
        hsum = hsum + ht
        if t == SEQ_NUM - 1:
            h_last = ht

    batch = batch_ref[0, 0, :]
    rows = i * BLK + lax.broadcasted_iota(jnp.int32, (BLK, B), 0)
    valid = (rows < N_NODES).astype(f32)
    seg_ids = lax.broadcasted_iota(jnp.int32, (BLK, B), 1)
    oh = (batch[:, None] == seg_ids).astype(f32) * valid
    cdims = (((0,), (0,)), ((), ()))
    acc_c[...] += lax.dot_general(oh, hc, cdims, preferred_element_type=f32)
    acc_h[...] += lax.dot_general(oh, hsum, cdims, preferred_element_type=f32)
    acc_n[...] += lax.dot_general(oh, jnp.ones((BLK, H), f32), cdims,
                                  preferred_element_type=f32)
    m_idx = (rows == idx_ref[...]).astype(f32)
    acc_o[...] += lax.dot_general(m_idx, h_last, cdims,
                                  preferred_element_type=f32)

    @pl.when(i == pl.num_programs(0) - 1)
    def _fin():
        cnt = jnp.maximum(acc_n[...], 1.0)
        ctx_c = acc_c[...] / cnt
        ctx_h = acc_h[...] / (cnt * float(SEQ_NUM))
        time_emb = (jnp.dot(oh43_ref[...], Wfc_ref[...],
                            preferred_element_type=f32) + bfc_ref[...])
        fake = (
            jnp.dot(jnp.tanh(ctx_c), Wout_ref[0], preferred_element_type=f32)
            + jnp.dot(jnp.tanh(ctx_h), Wout_ref[1], preferred_element_type=f32)
            + jnp.dot(jnp.tanh(time_emb), Wout_ref[2],
                      preferred_element_type=f32)
            + jnp.dot(jnp.tanh(semb_ref[...]), Wout_ref[3],
                      preferred_element_type=f32)
            + jnp.dot(jnp.tanh(z_ref[...]), Wout_ref[4],
                      preferred_element_type=f32)
            + bout_ref[...] + acc_o[...])
        out_ref[...] = fake


def _tc_fuse(agg_all, batch3, idx2, oh43, s_emb, z,
             W_g, b_g, W_st, b_st, W_fc, b_fc, Wout5, b_out):
    whole = lambda *shape: pl.BlockSpec(shape, lambda i: tuple(0 for _ in shape))
    coff = NPAD // BLK
    return pl.pallas_call(
        _tc_body,
        grid=(GRID,),
        in_specs=[
            pl.BlockSpec((BLK, F), lambda i: (i, 0)),
            pl.BlockSpec((BLK, F), lambda i: (i + coff, 0)),
            pl.BlockSpec((1, 1, BLK), lambda i: (i, 0, 0)),
            whole(1, B),
            whole(B, 43),
            whole(B, H),
            whole(B, H),
            whole(H, H),
            whole(1, H),
            whole(SEQ_LEN, H),
            whole(1, H),
            whole(43, H),
            whole(1, H),
            whole(5, H, H),
            whole(1, H),
        ],
        out_specs=pl.BlockSpec((B, H), lambda i: (0, 0)),
        out_shape=jax.ShapeDtypeStruct((B, H), jnp.float32),
        scratch_shapes=[
            pltpu.VMEM((B, H), jnp.float32),
            pltpu.VMEM((B, H), jnp.float32),
            pltpu.VMEM((B, H), jnp.float32),
            pltpu.VMEM((B, H), jnp.float32),
        ],
    )(agg_all, agg_all, batch3, idx2, oh43, s_emb, z,
      W_g, b_g, W_st, b_st, W_fc, b_fc, Wout5, b_out)


def kernel(graph_c, graph_h, edge_index, edge_attr, batch_vec,
           time_dayofweek, time_hour, time_minute, s_index, index,
           S_emb, W_g, b_g, W_st, b_st, W_fc, b_fc, W_out, b_out):
    xh = graph_h.reshape(N_NODES, F)
    xc = jnp.pad(graph_c, ((0, 0), (0, F - H)))
    x_split = jnp.concatenate([xh, xc], axis=0)
    src = edge_index[0]
    dst = edge_index[1]

    agg_all, s_emb = _sc_segment_sum(x_split, src, dst, edge_attr,
                                     s_index, S_emb)

    week = jax.nn.one_hot(time_dayofweek, 7, dtype=jnp.float32)
    hour = jax.nn.one_hot(time_hour, 24, dtype=jnp.float32)
    minute = jax.nn.one_hot(time_minute, 12, dtype=jnp.float32)
    oh43 = jnp.concatenate([week, hour, minute], axis=1)
    z = jax.random.uniform(jax.random.key(42), (B, H), dtype=jnp.float32)

    batch3 = jnp.pad(batch_vec, (0, NPAD - N_NODES),
                     constant_values=0).reshape(GRID, 1, BLK)
    idx2 = index.reshape(1, B)

    return _tc_fuse(agg_all, batch3, idx2, oh43, s_emb, z,
                    W_g, b_g.reshape(1, H), W_st, b_st.reshape(1, H),
                    W_fc, b_fc.reshape(1, H), W_out.reshape(5, H, H),
                    b_out.reshape(1, H))


# speedup vs baseline: 1.0235x; 1.0235x over previous
import functools

import jax
import jax.numpy as jnp
from jax import lax
from jax.experimental import pallas as pl
from jax.experimental.pallas import tpu as pltpu
from jax.experimental.pallas import tpu_sc as plsc

N_NODES = 10000
E_EDGES = 320000
H = 128
SEQ_NUM = 12
SEQ_LEN = 12
B = 16
F = 144
NSC = 2
NSUB = 16
EPT = E_EDGES // NSUB
EB = 4000
NBLK = EPT // EB
CK = 80
NCHUNK = EB // CK
NBUF = 2
NPAD = 10240
RPT = NPAD // NSUB
NVR = F // 16


_GD = lax.GatherDimensionNumbers(
    offset_dims=(), collapsed_slice_dims=(0,), start_index_map=(0,))


def _lane_bcast(vec, e):
    idx = jnp.full((16, 1), e, jnp.int32)
    return lax.gather(vec, idx, _GD, (1,),
                      mode=lax.GatherScatterMode.PROMISE_IN_BOUNDS)


def _sc_segment_sum(x_split, src, dst, w, s_index, S_emb):
    mesh = plsc.VectorSubcoreMesh(core_axis_name="c", subcore_axis_name="s")

    @functools.partial(
        pl.kernel,
        mesh=mesh,
        compiler_params=pltpu.CompilerParams(use_tc_tiling_on_sc=False),
        out_type=[
            jax.ShapeDtypeStruct((NSC * NPAD, F), jnp.float32),
            jax.ShapeDtypeStruct((B, H), jnp.float32),
        ],
        scratch_types=[
            pltpu.VMEM_SHARED((NPAD, F), jnp.float32),
            pltpu.VMEM((EB,), jnp.int32),
            pltpu.VMEM((EB,), jnp.int32),
            pltpu.VMEM((EB,), jnp.float32),
            [pltpu.VMEM((CK,), jnp.int32) for _ in range(NBUF)],
            [pltpu.VMEM((CK,), jnp.int32) for _ in range(NBUF)],
            [pltpu.VMEM((CK, F), jnp.float32) for _ in range(NBUF)],
            pltpu.VMEM((B,), jnp.int32),
            pltpu.VMEM((B, H), jnp.float32),
            pltpu.SemaphoreType.DMA,
            [pltpu.SemaphoreType.DMA for _ in range(NBUF)],
            [pltpu.SemaphoreType.DMA for _ in range(NBUF)],
        ],
    )
    def k(x_hbm, src_hbm, dst_hbm, w_hbm, sidx_hbm, semb_hbm,
          out_hbm, sout_hbm,
          acc, srcv, dstv, wv, gidxs, dchs, rowss,
          sibuf, sebuf, sem, gsems, ssems):
        cid = lax.axis_index("c")
        sid = lax.axis_index("s")
        rows0 = rowss[0]

        zv = jnp.zeros((16,), jnp.float32)
        for r in range(CK):
            for j in range(NVR):
                rows0[r, pl.ds(j * 16, 16)] = zv
        for t in range(RPT // CK):
            pltpu.sync_copy(rows0, acc.at[pl.ds(sid * RPT + t * CK, CK)])
        plsc.subcore_barrier()

        gath_off = cid * N_NODES

        def prep(c, gidx, dch):
            cb = c * CK
            for j in range(CK // 16):
                sl = pl.ds(cb + j * 16, 16)
                gidx[pl.ds(j * 16, 16)] = srcv[sl] + gath_off
                dch[pl.ds(j * 16, 16)] = dstv[sl]

        def scale(c, rows):
            cb = c * CK
            for g in range(CK // 16):
                w16 = wv[pl.ds(cb + g * 16, 16)]
                for e in range(16):
                    wvec = _lane_bcast(w16, e)
                    for j in range(NVR):
                        sl = pl.ds(j * 16, 16)
                        rows[g * 16 + e, sl] = rows[g * 16 + e, sl] * wvec

        def block_body(b, carry):
            ebase = sid * EPT + b * EB
            pltpu.sync_copy(src_hbm.at[pl.ds(ebase, EB)], srcv)
            pltpu.sync_copy(dst_hbm.at[pl.ds(ebase, EB)], dstv)
            pltpu.sync_copy(w_hbm.at[pl.ds(ebase, EB)], wv)

            def ring_body(i, carry2):
                started = (b > 0) | (i > 0)
                gs = []
                for u in range(NBUF):
                    c = NBUF * i + u

                    @pl.when(started)
                    def _(u=u):
                        pltpu.make_async_copy(
                            rowss[u], acc.at[dchs[u]], ssems[u]).wait()

                    prep(c, gidxs[u], dchs[u])
                    gs.append(pltpu.async_copy(
                        x_hbm.at[gidxs[u]], rowss[u], gsems[u]))
                for u in range(NBUF):
                    c = NBUF * i + u
                    gs[u].wait()
                    scale(c, rowss[u])
                    pltpu.async_copy(rowss[u], acc.at[dchs[u]], ssems[u],
                                     add=True)
                return carry2

            lax.fori_loop(0, NCHUNK // NBUF, ring_body, 0)
            return carry

        lax.fori_loop(0, NBLK, block_body, 0)
        for u in range(NBUF):
            pltpu.make_async_copy(rowss[u], acc.at[dchs[u]], ssems[u]).wait()
        plsc.subcore_barrier()

        out_off = cid * NPAD
        for t in range(RPT // CK):
            rb = sid * RPT + t * CK
            pltpu.sync_copy(acc.at[pl.ds(rb, CK)], rows0)
            pltpu.sync_copy(rows0, out_hbm.at[pl.ds(out_off + rb, CK)])

        @pl.when(jnp.logical_and(cid == 0, sid == 0))
        def _():
            pltpu.sync_copy(sidx_hbm, sibuf)
            pltpu.async_copy(semb_hbm.at[sibuf], sebuf, sem).wait()
            pltpu.sync_copy(sebuf, sout_hbm)

    return k(x_split, src, dst, w, s_index, S_emb)


BLK = 512
GRID = NPAD // BLK


def _tc_body(agghp_ref, aggcp_ref, batch_ref, idx_ref, oh43_ref, semb_ref,
             z_ref, Wg_ref, bg_ref, Wst_ref, bst_ref, Wfc_ref, bfc_ref,
             Wout_ref, bout_ref, out_ref, acc_c, acc_h, acc_o, acc_n):
    i = pl.program_id(0)

    @pl.when(i == 0)
    def _init():
        acc_c[...] = jnp.zeros_like(acc_c)
        acc_h[...] = jnp.zeros_like(acc_h)
        acc_o[...] = jnp.zeros_like(acc_o)
        acc_n[...] = jnp.zeros_like(acc_n)

    f32 = jnp.float32
    hc = jnp.tanh(
        jnp.dot(aggcp_ref[:, :H], Wg_ref[...], preferred_element_type=f32)
        + bg_ref[...])
    aggh = agghp_ref[...]
    Wst = Wst_ref[...]
    bst = bst_ref[...]
    hsum = jnp.zeros((BLK, H), f32)
    h_last = None
    for t in range(SEQ_NUM):
        ht = jnp.tanh(
            jnp.dot(aggh[:, t * SEQ_LEN:(t + 1) * SEQ_LEN], Wst,
                    preferred_element_type=f32) + bst)
        hsum = hsum + ht
        if t == SEQ_NUM - 1:
            h_last = ht

    batch = batch_ref[0, 0, :]
    rows = i * BLK + lax.broadcasted_iota(jnp.int32, (BLK, B), 0)
    valid = (rows < N_NODES).astype(f32)
    seg_ids = lax.broadcasted_iota(jnp.int32, (BLK, B), 1)
    oh = (batch[:, None] == seg_ids).astype(f32) * valid
    cdims = (((0,), (0,)), ((), ()))
    acc_c[...] += lax.dot_general(oh, hc, cdims, preferred_element_type=f32)
    acc_h[...] += lax.dot_general(oh, hsum, cdims, preferred_element_type=f32)
    acc_n[...] += lax.dot_general(oh, jnp.ones((BLK, H), f32), cdims,
                                  preferred_element_type=f32)
    m_idx = (rows == idx_ref[...]).astype(f32)
    acc_o[...] += lax.dot_general(m_idx, h_last, cdims,
                                  preferred_element_type=f32)

    @pl.when(i == pl.num_programs(0) - 1)
    def _fin():
        cnt = jnp.maximum(acc_n[...], 1.0)
        ctx_c = acc_c[...] / cnt
        ctx_h = acc_h[...] / (cnt * float(SEQ_NUM))
        time_emb = (jnp.dot(oh43_ref[...], Wfc_ref[...],
                            preferred_element_type=f32) + bfc_ref[...])
        fake = (
            jnp.dot(jnp.tanh(ctx_c), Wout_ref[0], preferred_element_type=f32)
            + jnp.dot(jnp.tanh(ctx_h), Wout_ref[1], preferred_element_type=f32)
            + jnp.dot(jnp.tanh(time_emb), Wout_ref[2],
                      preferred_element_type=f32)
            + jnp.dot(jnp.tanh(semb_ref[...]), Wout_ref[3],
                      preferred_element_type=f32)
            + jnp.dot(jnp.tanh(z_ref[...]), Wout_ref[4],
                      preferred_element_type=f32)
            + bout_ref[...] + acc_o[...])
        out_ref[...] = fake


def _tc_fuse(agg_all, batch3, idx2, oh43, s_emb, z,
             W_g, b_g, W_st, b_st, W_fc, b_fc, Wout5, b_out):
    whole = lambda *shape: pl.BlockSpec(shape, lambda i: tuple(0 for _ in shape))
    coff = NPAD // BLK
    return pl.pallas_call(
        _tc_body,
        grid=(GRID,),
        in_specs=[
            pl.BlockSpec((BLK, F), lambda i: (i, 0)),
            pl.BlockSpec((BLK, F), lambda i: (i + coff, 0)),
            pl.BlockSpec((1, 1, BLK), lambda i: (i, 0, 0)),
            whole(1, B),
            whole(B, 43),
            whole(B, H),
            whole(B, H),
            whole(H, H),
            whole(1, H),
            whole(SEQ_LEN, H),
            whole(1, H),
            whole(43, H),
            whole(1, H),
            whole(5, H, H),
            whole(1, H),
        ],
        out_specs=pl.BlockSpec((B, H), lambda i: (0, 0)),
        out_shape=jax.ShapeDtypeStruct((B, H), jnp.float32),
        scratch_shapes=[
            pltpu.VMEM((B, H), jnp.float32),
            pltpu.VMEM((B, H), jnp.float32),
            pltpu.VMEM((B, H), jnp.float32),
            pltpu.VMEM((B, H), jnp.float32),
        ],
    )(agg_all, agg_all, batch3, idx2, oh43, s_emb, z,
      W_g, b_g, W_st, b_st, W_fc, b_fc, Wout5, b_out)


def kernel(graph_c, graph_h, edge_index, edge_attr, batch_vec,
           time_dayofweek, time_hour, time_minute, s_index, index,
           S_emb, W_g, b_g, W_st, b_st, W_fc, b_fc, W_out, b_out):
    xh = graph_h.reshape(N_NODES, F)
    xc = jnp.pad(graph_c, ((0, 0), (0, F - H)))
    x_split = jnp.concatenate([xh, xc], axis=0)
    src = edge_index[0]
    dst = edge_index[1]

    agg_all, s_emb = _sc_segment_sum(x_split, src, dst, edge_attr,
                                     s_index, S_emb)

    week = jax.nn.one_hot(time_dayofweek, 7, dtype=jnp.float32)
    hour = jax.nn.one_hot(time_hour, 24, dtype=jnp.float32)
    minute = jax.nn.one_hot(time_minute, 12, dtype=jnp.float32)
    oh43 = jnp.concatenate([week, hour, minute], axis=1)
    z = jax.random.uniform(jax.random.key(42), (B, H), dtype=jnp.float32)

    batch3 = jnp.pad(batch_vec, (0, NPAD - N_NODES),
                     constant_values=0).reshape(GRID, 1, BLK)
    idx2 = index.reshape(1, B)

    return _tc_fuse(agg_all, batch3, idx2, oh43, s_emb, z,
                    W_g, b_g.reshape(1, H), W_st, b_st.reshape(1, H),
                    W_fc, b_fc.reshape(1, H), W_out.reshape(5, H, H),
                    b_out.reshape(1, H))
